# baseline (speedup 1.0000x reference)
import functools

import jax
import jax.numpy as jnp
from jax import lax
from jax.experimental import pallas as pl
from jax.experimental.pallas import tpu as pltpu
from jax.experimental.pallas import tpu_sc as plsc

BS = 4096
N_DIST = 20
N_FEAT = 26
DIM = 32
NH = 128

ROWS = BS * N_DIST
PAIRS = ROWS // 2
NCHUNK = 13
TOTAL_IDX = ROWS * N_FEAT

NW = 32
IDX_PER_W = TOTAL_IDX // NW
GL = 128
K = 8
NCH = IDX_PER_W // (K * GL)


def _sc_gather(idx4, table):
    mesh = plsc.VectorSubcoreMesh(core_axis_name="c", subcore_axis_name="s")

    @functools.partial(
        pl.kernel,
        out_type=jax.ShapeDtypeStruct((NW, NCH, K, GL, DIM), jnp.float32),
        mesh=mesh,
        compiler_params=pltpu.CompilerParams(use_tc_tiling_on_sc=False),
        scratch_types=[
            pltpu.VMEM((K, GL), jnp.int32),
            pltpu.VMEM((K, GL, DIM), jnp.float32),
            pltpu.SemaphoreType.DMA,
        ],
    )
    def body(idx_hbm, table_hbm, out_hbm, idx_v, rows_v, sem):
        wid = lax.axis_index("s") * 2 + lax.axis_index("c")

        @pl.loop(0, NCH)
        def _chunk(ch):
            pltpu.sync_copy(idx_hbm.at[wid, ch], idx_v)
            descs = [
                pltpu.async_copy(table_hbm.at[idx_v.at[j]], rows_v.at[j], sem)
                for j in range(K)
            ]
            for d in descs:
                d.wait()
            pltpu.sync_copy(rows_v, out_hbm.at[wid, ch])

    return body(idx4, table)


def _tc_score(g, x, idx4d, weo, b2):
    PB = 640
    BB = PB // 10

    def body(g_ref, x_ref, idx_ref, w_ref, b_ref, oe_ref, oo_ref):
        gb = g_ref[...].astype(jnp.bfloat16)
        wb = w_ref[...].astype(jnp.bfloat16)
        acc = jnp.zeros((PB, 2 * NH), jnp.float32)
        for c in range(NCHUNK):
            acc += jnp.dot(gb[c], wb[c], preferred_element_type=jnp.float32)
        h = jnp.tanh(acc + b_ref[...])
        xb3 = jnp.broadcast_to(x_ref[...][:, None, :], (BB, 10, NH))
        he3 = h[:, :NH].reshape(BB, 10, NH)
        ho3 = h[:, NH:].reshape(BB, 10, NH)
        de = jnp.sum(he3 * xb3, axis=-1)
        do = jnp.sum(ho3 * xb3, axis=-1)
        idx = idx_ref[...]
        me = jnp.all(idx[:, :, 0, :] == 0, axis=-1)
        mo = jnp.all(idx[:, :, 1, :] == 0, axis=-1)
        oe_ref[...] = jnp.where(me, -jnp.inf, de)
        oo_ref[...] = jnp.where(mo, -jnp.inf, do)

    return pl.pallas_call(
        body,
        grid=(PAIRS // PB,),
        in_specs=[
            pl.BlockSpec((NCHUNK, PB, NH), lambda i: (0, i, 0)),
            pl.BlockSpec((BB, NH), lambda i: (i, 0)),
            pl.BlockSpec((BB, 10, 2, N_FEAT), lambda i: (i, 0, 0, 0)),
            pl.BlockSpec((NCHUNK, NH, 2 * NH), lambda i: (0, 0, 0)),
            pl.BlockSpec((1, 2 * NH), lambda i: (0, 0)),
        ],
        out_specs=[
            pl.BlockSpec((BB, 10), lambda i: (i, 0)),
            pl.BlockSpec((BB, 10), lambda i: (i, 0)),
        ],
        out_shape=[
            jax.ShapeDtypeStruct((BS, 10), jnp.float32),
            jax.ShapeDtypeStruct((BS, 10), jnp.float32),
        ],
    )(g, x, idx4d, weo, b2)


def kernel(x, _input, table, W, b):
    idxp = _input.reshape(PAIRS, NCHUNK, 4).transpose(1, 0, 2)
    idx4 = idxp.reshape(NW, NCH, K, GL)
    g = _sc_gather(idx4, table).reshape(NCHUNK, PAIRS, NH)

    zeros = jnp.zeros_like(W)
    weo = jnp.concatenate(
        [jnp.concatenate([W, zeros], axis=0).reshape(NCHUNK, NH, NH),
         jnp.concatenate([zeros, W], axis=0).reshape(NCHUNK, NH, NH)],
        axis=-1,
    )
    b2 = jnp.concatenate([b, b]).reshape(1, 2 * NH)
    idx4d = _input.reshape(BS, 10, 2, N_FEAT)

    oe, oo = _tc_score(g, x, idx4d, weo, b2)
    return jnp.stack([oe, oo], axis=-1).reshape(BS, N_DIST)


# speedup vs baseline: 1.4551x; 1.4551x over previous
import functools

import jax
import jax.numpy as jnp
from jax import lax
from jax.experimental import pallas as pl
from jax.experimental.pallas import tpu as pltpu
from jax.experimental.pallas import tpu_sc as plsc

BS = 4096
N_DIST = 20
N_FEAT = 26
DIM = 32
NH = 128

ROWS = BS * N_DIST
PAIRS = ROWS // 2
NCHUNK = 13
TOTAL_IDX = ROWS * N_FEAT

NW = 32
IDX_PER_W = TOTAL_IDX // NW
GL = 128
K = 13
NCH = IDX_PER_W // (K * GL)
PAIRS_PER_W = IDX_PER_W // 52
L = 16
NV = K * GL // L


def _dest_rows():
    import numpy as np
    r = np.arange(TOTAL_IDX, dtype=np.int64)
    p, rem = r // 52, r % 52
    c, j4 = rem // 4, rem % 4
    dest = c * (4 * PAIRS) + p * 4 + j4
    return jnp.asarray(dest.astype(np.int32).reshape(NW, NCH, K, GL))


def _sc_gather(idx3, table):
    mesh = plsc.VectorSubcoreMesh(core_axis_name="c", subcore_axis_name="s")

    @functools.partial(
        pl.kernel,
        out_type=jax.ShapeDtypeStruct((TOTAL_IDX, DIM), jnp.float32),
        mesh=mesh,
        compiler_params=pltpu.CompilerParams(use_tc_tiling_on_sc=False),
        scratch_types=[
            pltpu.VMEM((K, GL), jnp.int32),
            pltpu.VMEM((K, GL), jnp.int32),
            pltpu.VMEM((K * GL, DIM), jnp.float32),
            pltpu.SemaphoreType.DMA,
            pltpu.SemaphoreType.DMA,
        ],
    )
    def body(idx_hbm, oidx_hbm, table_hbm, out_hbm,
             idx_v, oidx_v, rows_v, gsem, ssem):
        wid = lax.axis_index("s") * 2 + lax.axis_index("c")

        @pl.loop(0, NCH)
        def _chunk(ch):
            pltpu.sync_copy(idx_hbm.at[wid, ch], idx_v)
            pltpu.sync_copy(oidx_hbm.at[wid, ch], oidx_v)
            gd = [
                pltpu.async_copy(
                    table_hbm.at[idx_v.at[j]],
                    rows_v.at[pl.ds(j * GL, GL)], gsem)
                for j in range(K)
            ]
            for d in gd:
                d.wait()
            sd = [
                pltpu.async_copy(
                    rows_v.at[pl.ds(j * GL, GL)],
                    out_hbm.at[oidx_v.at[j]], ssem)
                for j in range(K)
            ]
            for d in sd:
                d.wait()

    oidx = _dest_rows()
    return body(idx3, oidx, table)


def _tc_score(g, x, idx, weo, b2):
    PB = 640
    BB = PB // 10

    def body(g_ref, x_ref, idx_ref, w_ref, b_ref, oe_ref, oo_ref):
        gb = g_ref[...].astype(jnp.bfloat16)
        wb = w_ref[...].astype(jnp.bfloat16)
        acc = jnp.zeros((PB, 2 * NH), jnp.float32)
        for c in range(NCHUNK):
            acc += jnp.dot(gb[c], wb[c], preferred_element_type=jnp.float32)
        h = jnp.tanh(acc + b_ref[...])
        xb3 = jnp.broadcast_to(x_ref[...][:, None, :], (BB, 10, NH))
        he3 = h[:, :NH].reshape(BB, 10, NH)
        ho3 = h[:, NH:].reshape(BB, 10, NH)
        de = jnp.sum(he3 * xb3, axis=-1)
        do = jnp.sum(ho3 * xb3, axis=-1)
        az = jnp.all(idx_ref[...] == 0, axis=-1).astype(jnp.float32)
        dsel = lax.broadcasted_iota(jnp.int32, (N_DIST, 10), 0)
        ksel = lax.broadcasted_iota(jnp.int32, (N_DIST, 10), 1)
        se = (dsel == 2 * ksel).astype(jnp.float32)
        so = (dsel == 2 * ksel + 1).astype(jnp.float32)
        me = jnp.dot(az, se, preferred_element_type=jnp.float32) > 0.5
        mo = jnp.dot(az, so, preferred_element_type=jnp.float32) > 0.5
        oe_ref[...] = jnp.where(me, -jnp.inf, de)
        oo_ref[...] = jnp.where(mo, -jnp.inf, do)

    return pl.pallas_call(
        body,
        grid=(PAIRS // PB,),
        in_specs=[
            pl.BlockSpec((NCHUNK, PB, NH), lambda i: (0, i, 0)),
            pl.BlockSpec((BB, NH), lambda i: (i, 0)),
            pl.BlockSpec((BB, N_DIST, N_FEAT), lambda i: (i, 0, 0)),
            pl.BlockSpec((NCHUNK, NH, 2 * NH), lambda i: (0, 0, 0)),
            pl.BlockSpec((1, 2 * NH), lambda i: (0, 0)),
        ],
        out_specs=[
            pl.BlockSpec((BB, 10), lambda i: (i, 0)),
            pl.BlockSpec((BB, 10), lambda i: (i, 0)),
        ],
        out_shape=[
            jax.ShapeDtypeStruct((BS, 10), jnp.float32),
            jax.ShapeDtypeStruct((BS, 10), jnp.float32),
        ],
    )(g, x, idx, weo, b2)


def kernel(x, _input, table, W, b):
    idx3 = _input.reshape(NW, NCH, K, GL)
    g = _sc_gather(idx3, table).reshape(NCHUNK, PAIRS, NH)

    zeros = jnp.zeros_like(W)
    weo = jnp.concatenate(
        [jnp.concatenate([W, zeros], axis=0).reshape(NCHUNK, NH, NH),
         jnp.concatenate([zeros, W], axis=0).reshape(NCHUNK, NH, NH)],
        axis=-1,
    )
    b2 = jnp.concatenate([b, b]).reshape(1, 2 * NH)

    oe, oo = _tc_score(g, x, _input, weo, b2)
    return jnp.stack([oe, oo], axis=-1).reshape(BS, N_DIST)
